# Initial kernel scaffold; baseline (speedup 1.0000x reference)
import jax
import jax.numpy as jnp
from jax.experimental import pallas as pl


def kernel(height_ids, width_ids, embed):
    raise NotImplementedError("write your pallas kernel here")


# speedup vs baseline: 6.1370x; 6.1370x over previous
import functools

import jax
import jax.numpy as jnp
from jax import lax
from jax.experimental import pallas as pl
from jax.experimental.pallas import tpu as pltpu
from jax.experimental.pallas import tpu_sc as plsc

MAX_W = 256
DIM = 128
LANES = 16

_info = plsc.get_sparse_core_info()
NC = _info.num_cores
NS = _info.num_subcores
NW = NC * NS

CHUNK = 128


def _make_kernel(B):
    b_per_w = B // NW
    n_chunks = b_per_w // CHUNK
    mesh = plsc.VectorSubcoreMesh(core_axis_name="c", subcore_axis_name="s")

    @functools.partial(
        pl.kernel,
        out_type=jax.ShapeDtypeStruct((B, DIM), jnp.float32),
        mesh=mesh,
        scratch_types=[
            pltpu.VMEM((b_per_w,), jnp.int32),
            pltpu.VMEM((b_per_w,), jnp.int32),
            pltpu.VMEM((CHUNK,), jnp.int32),
            pltpu.VMEM((CHUNK, DIM), jnp.float32),
            pltpu.SemaphoreType.DMA,
        ],
    )
    def body(h_hbm, w_hbm, table_hbm, out_hbm, h_v, w_v, idx_v, rows_v, sem):
        wid = lax.axis_index("s") * NC + lax.axis_index("c")
        base = wid * b_per_w
        pltpu.sync_copy(h_hbm.at[pl.ds(base, b_per_w)], h_v)
        pltpu.sync_copy(w_hbm.at[pl.ds(base, b_per_w)], w_v)

        def chunk_step(j, carry):
            off = j * CHUNK
            for i in range(CHUNK // LANES):
                s = off + i * LANES
                idx_v[pl.ds(i * LANES, LANES)] = (
                    h_v[pl.ds(s, LANES)] * MAX_W + w_v[pl.ds(s, LANES)]
                )
            pltpu.async_copy(table_hbm.at[idx_v], rows_v, sem).wait()
            pltpu.sync_copy(rows_v, out_hbm.at[pl.ds(base + off, CHUNK)])
            return carry

        lax.fori_loop(0, n_chunks, chunk_step, 0)

    return body


def kernel(height_ids, width_ids, embed):
    n, t = height_ids.shape
    B = n * t
    h = height_ids.reshape(B).astype(jnp.int32)
    w = width_ids.reshape(B).astype(jnp.int32)
    out = _make_kernel(B)(h, w, embed)
    return out.reshape(n, t, DIM)


# speedup vs baseline: 8.7859x; 1.4316x over previous
import functools

import jax
import jax.numpy as jnp
from jax import lax
from jax.experimental import pallas as pl
from jax.experimental.pallas import tpu as pltpu
from jax.experimental.pallas import tpu_sc as plsc

MAX_W = 256
DIM = 128
LANES = 16

_info = plsc.get_sparse_core_info()
NC = _info.num_cores
NS = _info.num_subcores
NW = NC * NS

CHUNK = 128


def _make_kernel(B):
    b_per_w = B // NW
    n_chunks = b_per_w // CHUNK
    mesh = plsc.VectorSubcoreMesh(core_axis_name="c", subcore_axis_name="s")

    @functools.partial(
        pl.kernel,
        out_type=jax.ShapeDtypeStruct((B, DIM), jnp.float32),
        mesh=mesh,
        scratch_types=[
            pltpu.VMEM((b_per_w,), jnp.int32),
            pltpu.VMEM((b_per_w,), jnp.int32),
            pltpu.VMEM((2, CHUNK), jnp.int32),
            pltpu.VMEM((2, CHUNK, DIM), jnp.float32),
            pltpu.SemaphoreType.DMA,
            pltpu.SemaphoreType.DMA,
        ],
    )
    def body(h_hbm, w_hbm, table_hbm, out_hbm, h_v, w_v, idx_v, rows_v,
             sem0, sem1):
        sems = (sem0, sem1)
        wid = lax.axis_index("s") * NC + lax.axis_index("c")
        base = wid * b_per_w
        pltpu.sync_copy(h_hbm.at[pl.ds(base, b_per_w)], h_v)
        pltpu.sync_copy(w_hbm.at[pl.ds(base, b_per_w)], w_v)

        def fire(j, b):
            off = j * CHUNK
            for i in range(CHUNK // LANES):
                s = off + i * LANES
                idx_v[b, pl.ds(i * LANES, LANES)] = (
                    h_v[pl.ds(s, LANES)] * MAX_W + w_v[pl.ds(s, LANES)]
                )
            pltpu.async_copy(table_hbm.at[idx_v.at[b]], rows_v.at[b], sems[b])

        fire(0, 0)
        fire(1, 1)

        def group(g, carry):
            for b in range(2):
                j = g * 2 + b
                pltpu.make_async_copy(
                    table_hbm.at[idx_v.at[b]], rows_v.at[b], sems[b]
                ).wait()
                pltpu.sync_copy(
                    rows_v.at[b], out_hbm.at[pl.ds(base + j * CHUNK, CHUNK)]
                )
                jn = j + 2

                @pl.when(jn < n_chunks)
                def _():
                    fire(jn, b)

            return carry

        lax.fori_loop(0, n_chunks // 2, group, 0)

    return body


def kernel(height_ids, width_ids, embed):
    n, t = height_ids.shape
    B = n * t
    h = height_ids.reshape(B).astype(jnp.int32)
    w = width_ids.reshape(B).astype(jnp.int32)
    out = _make_kernel(B)(h, w, embed)
    return out.reshape(n, t, DIM)


# speedup vs baseline: 8.8131x; 1.0031x over previous
import functools

import jax
import jax.numpy as jnp
from jax import lax
from jax.experimental import pallas as pl
from jax.experimental.pallas import tpu as pltpu
from jax.experimental.pallas import tpu_sc as plsc

MAX_W = 256
DIM = 128
LANES = 16

_info = plsc.get_sparse_core_info()
NC = _info.num_cores
NS = _info.num_subcores
NW = NC * NS

CHUNK = 128


def _make_kernel(B):
    b_per_w = B // NW
    n_chunks = b_per_w // CHUNK
    mesh = plsc.VectorSubcoreMesh(core_axis_name="c", subcore_axis_name="s")

    NBUF = 4
    LAG = 2

    @functools.partial(
        pl.kernel,
        out_type=jax.ShapeDtypeStruct((B, DIM), jnp.float32),
        mesh=mesh,
        scratch_types=[
            pltpu.VMEM((b_per_w,), jnp.int32),
            pltpu.VMEM((b_per_w,), jnp.int32),
            pltpu.VMEM((NBUF, CHUNK), jnp.int32),
            pltpu.VMEM((NBUF, CHUNK, DIM), jnp.float32),
            pltpu.SemaphoreType.DMA,
            pltpu.SemaphoreType.DMA,
            pltpu.SemaphoreType.DMA,
            pltpu.SemaphoreType.DMA,
            pltpu.SemaphoreType.DMA,
            pltpu.SemaphoreType.DMA,
            pltpu.SemaphoreType.DMA,
            pltpu.SemaphoreType.DMA,
        ],
    )
    def body(h_hbm, w_hbm, table_hbm, out_hbm, h_v, w_v, idx_v, rows_v,
             g0, g1, g2, g3, w0, w1, w2, w3):
        sem_g = (g0, g1, g2, g3)
        sem_w = (w0, w1, w2, w3)
        wid = lax.axis_index("s") * NC + lax.axis_index("c")
        base = wid * b_per_w
        pltpu.sync_copy(h_hbm.at[pl.ds(base, b_per_w)], h_v)
        pltpu.sync_copy(w_hbm.at[pl.ds(base, b_per_w)], w_v)

        def fire_gather(j, b):
            off = j * CHUNK
            for i in range(CHUNK // LANES):
                s = off + i * LANES
                idx_v[b, pl.ds(i * LANES, LANES)] = (
                    h_v[pl.ds(s, LANES)] * MAX_W + w_v[pl.ds(s, LANES)]
                )
            pltpu.async_copy(table_hbm.at[idx_v.at[b]], rows_v.at[b], sem_g[b])

        def wait_gather(b):
            pltpu.make_async_copy(
                table_hbm.at[idx_v.at[b]], rows_v.at[b], sem_g[b]
            ).wait()

        def fire_wb(j, b):
            pltpu.async_copy(
                rows_v.at[b], out_hbm.at[pl.ds(base + j * CHUNK, CHUNK)],
                sem_w[b],
            )

        def wait_wb(j, b):
            pltpu.make_async_copy(
                rows_v.at[b], out_hbm.at[pl.ds(base + j * CHUNK, CHUNK)],
                sem_w[b],
            ).wait()

        n_groups = (n_chunks + LAG + NBUF - 1) // NBUF

        def group(g, carry):
            for b in range(NBUF):
                j = g * NBUF + b

                @pl.when((j >= NBUF) & (j < n_chunks))
                def _():
                    wait_wb(j - NBUF, b)

                @pl.when(j < n_chunks)
                def _():
                    fire_gather(j, b)

                j2 = j - LAG
                b2 = (b - LAG) % NBUF

                @pl.when((j2 >= 0) & (j2 < n_chunks))
                def _():
                    wait_gather(b2)
                    fire_wb(j2, b2)

            return carry

        lax.fori_loop(0, n_groups, group, 0)

        for b in range(NBUF):
            wait_wb(n_chunks - NBUF + b, b)

    return body


def kernel(height_ids, width_ids, embed):
    n, t = height_ids.shape
    B = n * t
    h = height_ids.reshape(B).astype(jnp.int32)
    w = width_ids.reshape(B).astype(jnp.int32)
    out = _make_kernel(B)(h, w, embed)
    return out.reshape(n, t, DIM)


# speedup vs baseline: 8.8407x; 1.0031x over previous
import functools

import jax
import jax.numpy as jnp
from jax import lax
from jax.experimental import pallas as pl
from jax.experimental.pallas import tpu as pltpu
from jax.experimental.pallas import tpu_sc as plsc

MAX_W = 256
DIM = 128
LANES = 16

_info = plsc.get_sparse_core_info()
NC = _info.num_cores
NS = _info.num_subcores
NW = NC * NS

CHUNK = 128


def _make_kernel(B):
    b_per_w = B // NW
    n_chunks = b_per_w // CHUNK
    mesh = plsc.VectorSubcoreMesh(core_axis_name="c", subcore_axis_name="s")

    NBUF = 6
    ILAG = 2
    GLAG = 2

    @functools.partial(
        pl.kernel,
        out_type=jax.ShapeDtypeStruct((B, DIM), jnp.float32),
        mesh=mesh,
        scratch_types=[
            pltpu.VMEM((NBUF, CHUNK), jnp.int32),
            pltpu.VMEM((NBUF, CHUNK), jnp.int32),
            pltpu.VMEM((NBUF, CHUNK), jnp.int32),
            pltpu.VMEM((NBUF, CHUNK, DIM), jnp.float32),
        ] + [pltpu.SemaphoreType.DMA] * (3 * NBUF),
    )
    def body(h_hbm, w_hbm, table_hbm, out_hbm, h_v, w_v, idx_v, rows_v,
             *sems):
        sem_i = sems[0:NBUF]
        sem_g = sems[NBUF:2 * NBUF]
        sem_w = sems[2 * NBUF:3 * NBUF]
        wid = lax.axis_index("s") * NC + lax.axis_index("c")
        base = wid * b_per_w

        def fire_ids(j, b):
            pltpu.async_copy(
                h_hbm.at[pl.ds(base + j * CHUNK, CHUNK)], h_v.at[b], sem_i[b]
            )
            pltpu.async_copy(
                w_hbm.at[pl.ds(base + j * CHUNK, CHUNK)], w_v.at[b], sem_i[b]
            )

        def wait_ids(b):
            pltpu.make_async_copy(
                h_hbm.at[pl.ds(base, CHUNK)], h_v.at[b], sem_i[b]
            ).wait()
            pltpu.make_async_copy(
                w_hbm.at[pl.ds(base, CHUNK)], w_v.at[b], sem_i[b]
            ).wait()

        def fire_gather(b):
            for i in range(CHUNK // LANES):
                s = i * LANES
                idx_v[b, pl.ds(s, LANES)] = (
                    h_v[b, pl.ds(s, LANES)] * MAX_W + w_v[b, pl.ds(s, LANES)]
                )
            pltpu.async_copy(table_hbm.at[idx_v.at[b]], rows_v.at[b], sem_g[b])

        def wait_gather(b):
            pltpu.make_async_copy(
                table_hbm.at[idx_v.at[b]], rows_v.at[b], sem_g[b]
            ).wait()

        def fire_wb(j, b):
            pltpu.async_copy(
                rows_v.at[b], out_hbm.at[pl.ds(base + j * CHUNK, CHUNK)],
                sem_w[b],
            )

        def wait_wb(j, b):
            pltpu.make_async_copy(
                rows_v.at[b], out_hbm.at[pl.ds(base + j * CHUNK, CHUNK)],
                sem_w[b],
            ).wait()

        n_slots = n_chunks + ILAG + GLAG
        n_groups = (n_slots + NBUF - 1) // NBUF

        def group(g, carry):
            for b in range(NBUF):
                j = g * NBUF + b

                @pl.when((j >= NBUF) & (j < n_chunks))
                def _():
                    wait_wb(j - NBUF, b)

                @pl.when(j < n_chunks)
                def _():
                    fire_ids(j, b)

                j1 = j - ILAG
                b1 = (b - ILAG) % NBUF

                @pl.when((j1 >= 0) & (j1 < n_chunks))
                def _():
                    wait_ids(b1)
                    fire_gather(b1)

                j2 = j - ILAG - GLAG
                b2 = (b - ILAG - GLAG) % NBUF

                @pl.when((j2 >= 0) & (j2 < n_chunks))
                def _():
                    wait_gather(b2)
                    fire_wb(j2, b2)

            return carry

        lax.fori_loop(0, n_groups, group, 0)

        first = n_chunks - NBUF
        for k in range(NBUF):
            j = first + k
            wait_wb(j, j % NBUF)

    return body


def kernel(height_ids, width_ids, embed):
    n, t = height_ids.shape
    B = n * t
    h = height_ids.reshape(B).astype(jnp.int32)
    w = width_ids.reshape(B).astype(jnp.int32)
    out = _make_kernel(B)(h, w, embed)
    return out.reshape(n, t, DIM)
